# baseline (speedup 1.0000x reference)
import functools

import jax
import jax.numpy as jnp
from jax import lax
from jax.experimental import pallas as pl
from jax.experimental.pallas import tpu as pltpu
from jax.experimental.pallas import tpu_sc as plsc

DIM = 64
LANES = 16
CHUNK = 64
NEG_K = 5


def _sc_scores(u_idx, vp_idx, vn_idx, u_flat, v_flat, vocab):
    B = u_idx.shape[0]
    info = plsc.get_sparse_core_info()
    nc, ns = info.num_cores, info.num_subcores
    nw = nc * ns
    per_w = B // nw
    n_chunks = per_w // CHUNK
    n_grp = CHUNK // LANES
    mesh = plsc.VectorSubcoreMesh(core_axis_name="c", subcore_axis_name="s")

    @functools.partial(
        pl.kernel,
        mesh=mesh,
        compiler_params=pltpu.CompilerParams(needs_layout_passes=False),
        out_type=(
            jax.ShapeDtypeStruct((B,), jnp.float32),
            jax.ShapeDtypeStruct((B,), jnp.float32),
        ),
        scratch_types=[
            pltpu.VMEM((per_w,), jnp.int32),
            pltpu.VMEM((per_w,), jnp.int32),
            pltpu.VMEM((NEG_K * per_w,), jnp.int32),
            pltpu.VMEM((DIM * CHUNK,), jnp.float32),
            pltpu.VMEM((DIM * CHUNK,), jnp.float32),
            pltpu.VMEM((NEG_K * DIM * CHUNK,), jnp.float32),
            pltpu.VMEM((per_w,), jnp.float32),
            pltpu.VMEM((per_w,), jnp.float32),
            pltpu.SemaphoreType.DMA,
        ],
    )
    def k(u_idx_h, vp_idx_h, vn_idx_h, u_tab, v_tab, pos_out, neg_out,
          u_iv, vp_iv, vn_iv, ub, vpb, vnb, pos_sv, neg_sv, sem):
        wid = lax.axis_index("s") * nc + lax.axis_index("c")
        wbase = wid * per_w

        pltpu.sync_copy(u_idx_h.at[pl.ds(wbase, per_w)], u_iv)
        pltpu.sync_copy(vp_idx_h.at[pl.ds(wbase, per_w)], vp_iv)
        for n in range(NEG_K):
            pltpu.sync_copy(vn_idx_h.at[pl.ds(n * B + wbase, per_w)],
                            vn_iv.at[pl.ds(n * per_w, per_w)])

        for c in range(n_chunks):
            cb = c * CHUNK
            u_slice = u_iv.at[pl.ds(cb, CHUNK)]
            vp_slice = vp_iv.at[pl.ds(cb, CHUNK)]
            vn_slices = [vn_iv.at[pl.ds(n * per_w + cb, CHUNK)]
                         for n in range(NEG_K)]

            def fire_d(d, carry):
                seg = pl.multiple_of(d * vocab, 8)
                dof = pl.multiple_of(d * CHUNK, 8)
                pltpu.async_copy(
                    u_tab.at[pl.ds(seg, vocab)].at[u_slice],
                    ub.at[pl.ds(dof, CHUNK)], sem)
                pltpu.async_copy(
                    v_tab.at[pl.ds(seg, vocab)].at[vp_slice],
                    vpb.at[pl.ds(dof, CHUNK)], sem)
                for n in range(NEG_K):
                    nof = pl.multiple_of((n * DIM + d) * CHUNK, 8)
                    pltpu.async_copy(
                        v_tab.at[pl.ds(seg, vocab)].at[vn_slices[n]],
                        vnb.at[pl.ds(nof, CHUNK)], sem)
                return carry

            lax.fori_loop(0, DIM, fire_d, 0)

            pltpu.make_async_copy(
                u_tab.at[pl.ds(0, DIM * CHUNK)], ub, sem).wait()
            pltpu.make_async_copy(
                u_tab.at[pl.ds(0, DIM * CHUNK)], vpb, sem).wait()
            pltpu.make_async_copy(
                u_tab.at[pl.ds(0, NEG_K * DIM * CHUNK)], vnb, sem).wait()

            def grp_body(j, carry):
                jb = j * LANES

                def d_body(d, accs):
                    pos_acc, neg_acc = accs
                    off = d * CHUNK + jb
                    uv = ub[pl.ds(off, LANES)]
                    vv = vpb[pl.ds(off, LANES)]
                    ns = vnb[pl.ds(off, LANES)]
                    for n in range(1, NEG_K):
                        ns = ns + vnb[pl.ds(n * DIM * CHUNK + off, LANES)]
                    return (pos_acc + uv * vv, neg_acc + uv * ns)

                zf = jnp.zeros((LANES,), jnp.float32)
                pos_acc, neg_acc = lax.fori_loop(0, DIM, d_body, (zf, zf))
                pos_sv[pl.ds(cb + jb, LANES)] = pos_acc
                neg_sv[pl.ds(cb + jb, LANES)] = neg_acc
                return carry

            lax.fori_loop(0, n_grp, grp_body, 0)

        pltpu.sync_copy(pos_sv, pos_out.at[pl.ds(wbase, per_w)])
        pltpu.sync_copy(neg_sv, neg_out.at[pl.ds(wbase, per_w)])

    return k(u_idx, vp_idx, vn_idx, u_flat, v_flat)


def _tc_loss_body(pos_ref, neg_ref, bs_ref, out_ref):
    pos = pos_ref[...]
    neg = neg_ref[...]
    ls = jax.nn.log_sigmoid(pos) + jax.nn.log_sigmoid(-neg)
    out_ref[0, 0] = -jnp.sum(ls) / bs_ref[0].astype(jnp.float32)


def kernel(u_positive, v_positive, v_negative, batch_size, u_table, v_table):
    B = u_positive.shape[0]
    V = u_table.shape[0]
    u_idx = u_positive.astype(jnp.int32)
    vp_idx = v_positive.astype(jnp.int32)
    vn_idx = v_negative.astype(jnp.int32).T.reshape((-1,))

    u_flat = u_table.T.reshape((-1,))
    v_flat = v_table.T.reshape((-1,))

    pos_s, neg_s = _sc_scores(u_idx, vp_idx, vn_idx, u_flat, v_flat, V)

    rows = B // 128
    bs = jnp.asarray(batch_size, jnp.int32).reshape((1,))
    loss = pl.pallas_call(
        _tc_loss_body,
        out_shape=jax.ShapeDtypeStruct((1, 1), jnp.float32),
        in_specs=[
            pl.BlockSpec(memory_space=pltpu.VMEM),
            pl.BlockSpec(memory_space=pltpu.VMEM),
            pl.BlockSpec(memory_space=pltpu.SMEM),
        ],
        out_specs=pl.BlockSpec(memory_space=pltpu.SMEM),
    )(pos_s.reshape((rows, 128)), neg_s.reshape((rows, 128)), bs)
    return loss[0, 0]


# speedup vs baseline: 12.7778x; 12.7778x over previous
import functools

import jax
import jax.numpy as jnp
from jax import lax
from jax.experimental import pallas as pl
from jax.experimental.pallas import tpu as pltpu
from jax.experimental.pallas import tpu_sc as plsc

DIM = 64
LANES = 16
CHUNK = 64
NEG_K = 5
PACK_W = 4096


def _tc_pack_body(src_ref, out_ref):
    out_ref[:, 0:DIM] = src_ref[...].T


def _pack_table(table_t):
    V = table_t.shape[1]
    grid = (V + PACK_W - 1) // PACK_W
    return pl.pallas_call(
        _tc_pack_body,
        grid=(grid,),
        in_specs=[pl.BlockSpec((DIM, PACK_W), lambda j: (0, j))],
        out_specs=pl.BlockSpec((PACK_W, 2 * DIM), lambda j: (j, 0)),
        out_shape=jax.ShapeDtypeStruct((V, 2 * DIM), jnp.float32),
    )(table_t)


def _sc_scores(u_g, vp_g, vn_g, u_tab2, v_tab2):
    B = u_g.shape[0]
    info = plsc.get_sparse_core_info()
    nc, ns = info.num_cores, info.num_subcores
    nw = nc * ns
    per_w = B // nw
    n_chunks = per_w // CHUNK
    mesh = plsc.VectorSubcoreMesh(core_axis_name="c", subcore_axis_name="s")

    @functools.partial(
        pl.kernel,
        mesh=mesh,
        compiler_params=pltpu.CompilerParams(needs_layout_passes=False),
        out_type=(
            jax.ShapeDtypeStruct((B,), jnp.float32),
            jax.ShapeDtypeStruct((B,), jnp.float32),
        ),
        scratch_types=[
            pltpu.VMEM((per_w,), jnp.int32),
            pltpu.VMEM((per_w,), jnp.int32),
            pltpu.VMEM((NEG_K * per_w,), jnp.int32),
            pltpu.VMEM((2, CHUNK, 2 * DIM), jnp.float32),
            pltpu.VMEM((2, CHUNK, 2 * DIM), jnp.float32),
            pltpu.VMEM((2, NEG_K * CHUNK, 2 * DIM), jnp.float32),
            pltpu.VMEM((per_w,), jnp.float32),
            pltpu.VMEM((per_w,), jnp.float32),
            pltpu.SemaphoreType.DMA,
            pltpu.SemaphoreType.DMA,
        ],
    )
    def k(u_g_h, vp_g_h, vn_g_h, u_tab, v_tab,
          pos_out, neg_out,
          u_gv, vp_gv, vn_gv, u_b, vp_b, vn_b,
          pos_sv, neg_sv, sem0, sem1):
        wid = lax.axis_index("s") * nc + lax.axis_index("c")
        wbase = wid * per_w

        pltpu.sync_copy(u_g_h.at[pl.ds(wbase, per_w)], u_gv)
        pltpu.sync_copy(vp_g_h.at[pl.ds(wbase, per_w)], vp_gv)
        for n in range(NEG_K):
            pltpu.sync_copy(vn_g_h.at[pl.ds(n * B + wbase, per_w)],
                            vn_gv.at[pl.ds(n * per_w, per_w)])

        sems = (sem0, sem1)

        def fire(c):
            p = c % 2
            cb = c * CHUNK
            hs = [
                pltpu.async_copy(
                    u_tab.at[u_gv.at[pl.ds(cb, CHUNK)]], u_b.at[p], sems[p]),
                pltpu.async_copy(
                    v_tab.at[vp_gv.at[pl.ds(cb, CHUNK)]], vp_b.at[p], sems[p]),
            ]
            for n in range(NEG_K):
                hs.append(pltpu.async_copy(
                    v_tab.at[vn_gv.at[pl.ds(n * per_w + cb, CHUNK)]],
                    vn_b.at[p, pl.ds(n * CHUNK, CHUNK)], sems[p]))
            return hs

        pending = fire(0)
        iota16 = lax.iota(jnp.int32, LANES)

        for c in range(n_chunks):
            nxt = fire(c + 1) if c + 1 < n_chunks else []
            for h in pending:
                h.wait()
            pending = nxt
            p = c % 2
            cb = c * CHUNK

            def grp_body(g, carry):
                gb = g * LANES
                crow = (gb + iota16) * (2 * DIM)
                cu = crow
                cvp = crow
                cns = [crow + (n * CHUNK) * (2 * DIM)
                       for n in range(NEG_K)]
                zero16 = jnp.zeros((LANES,), jnp.int32)
                ub = u_b.at[p]
                vpb = vp_b.at[p]
                vnb = vn_b.at[p]

                def d_body(dj, accs):
                    pos_acc, neg_acc = accs
                    base = dj * 4
                    for t in range(4):
                        d = base + t
                        uv = plsc.load_gather(ub, [zero16, cu + d])
                        vv = plsc.load_gather(vpb, [zero16, cvp + d])
                        ns = plsc.load_gather(vnb, [zero16, cns[0] + d])
                        for n in range(1, NEG_K):
                            ns = ns + plsc.load_gather(
                                vnb, [zero16, cns[n] + d])
                        pos_acc = pos_acc + uv * vv
                        neg_acc = neg_acc + uv * ns
                    return (pos_acc, neg_acc)

                zf = jnp.zeros((LANES,), jnp.float32)
                pos_acc, neg_acc = lax.fori_loop(
                    0, DIM // 4, d_body, (zf, zf))
                pos_sv[pl.ds(cb + gb, LANES)] = pos_acc
                neg_sv[pl.ds(cb + gb, LANES)] = neg_acc
                return carry

            lax.fori_loop(0, CHUNK // LANES, grp_body, 0)

        pltpu.sync_copy(pos_sv, pos_out.at[pl.ds(wbase, per_w)])
        pltpu.sync_copy(neg_sv, neg_out.at[pl.ds(wbase, per_w)])

    return k(u_g, vp_g, vn_g, u_tab2, v_tab2)


def _tc_loss_body(pos_ref, neg_ref, bs_ref, out_ref):
    pos = pos_ref[...]
    neg = neg_ref[...]
    ls = jax.nn.log_sigmoid(pos) + jax.nn.log_sigmoid(-neg)
    out_ref[0, 0] = -jnp.sum(ls) / bs_ref[0].astype(jnp.float32)


def kernel(u_positive, v_positive, v_negative, batch_size, u_table, v_table):
    B = u_positive.shape[0]
    u_idx = u_positive.astype(jnp.int32)
    vp_idx = v_positive.astype(jnp.int32)
    vn_idx = v_negative.astype(jnp.int32).T.reshape((-1,))

    u_tab2 = _pack_table(u_table.T)
    v_tab2 = _pack_table(v_table.T)

    pos_s, neg_s = _sc_scores(u_idx, vp_idx, vn_idx, u_tab2, v_tab2)

    rows = B // 128
    bs = jnp.asarray(batch_size, jnp.int32).reshape((1,))
    loss = pl.pallas_call(
        _tc_loss_body,
        out_shape=jax.ShapeDtypeStruct((1, 1), jnp.float32),
        in_specs=[
            pl.BlockSpec(memory_space=pltpu.VMEM),
            pl.BlockSpec(memory_space=pltpu.VMEM),
            pl.BlockSpec(memory_space=pltpu.SMEM),
        ],
        out_specs=pl.BlockSpec(memory_space=pltpu.SMEM),
    )(pos_s.reshape((rows, 128)), neg_s.reshape((rows, 128)), bs)
    return loss[0, 0]


# speedup vs baseline: 13.8014x; 1.0801x over previous
import functools

import jax
import jax.numpy as jnp
from jax import lax
from jax.experimental import pallas as pl
from jax.experimental.pallas import tpu as pltpu
from jax.experimental.pallas import tpu_sc as plsc

DIM = 64
LANES = 16
CHUNK = 64
NEG_K = 5
PACK_W = 4096


def _tc_pack_body(src_ref, out_ref):
    x = src_ref[...]
    r = lax.broadcasted_iota(jnp.int32, (DIM, DIM), 0)
    c = lax.broadcasted_iota(jnp.int32, (DIM, DIM), 1)
    ident = (r == c).astype(jnp.float32)
    out_ref[:, 0:DIM] = lax.dot_general(
        x, ident, (((0,), (0,)), ((), ())),
        preferred_element_type=jnp.float32)


def _pack_table(table_t):
    V = table_t.shape[1]
    grid = (V + PACK_W - 1) // PACK_W
    return pl.pallas_call(
        _tc_pack_body,
        grid=(grid,),
        in_specs=[pl.BlockSpec((DIM, PACK_W), lambda j: (0, j))],
        out_specs=pl.BlockSpec((PACK_W, 2 * DIM), lambda j: (j, 0)),
        out_shape=jax.ShapeDtypeStruct((V, 2 * DIM), jnp.float32),
    )(table_t)


def _sc_scores(u_g, vp_g, vn_g, u_tab2, v_tab2):
    B = u_g.shape[0]
    info = plsc.get_sparse_core_info()
    nc, ns = info.num_cores, info.num_subcores
    nw = nc * ns
    per_w = B // nw
    n_chunks = per_w // CHUNK
    mesh = plsc.VectorSubcoreMesh(core_axis_name="c", subcore_axis_name="s")

    @functools.partial(
        pl.kernel,
        mesh=mesh,
        compiler_params=pltpu.CompilerParams(needs_layout_passes=False),
        out_type=(
            jax.ShapeDtypeStruct((B,), jnp.float32),
            jax.ShapeDtypeStruct((B,), jnp.float32),
        ),
        scratch_types=[
            pltpu.VMEM((per_w,), jnp.int32),
            pltpu.VMEM((per_w,), jnp.int32),
            pltpu.VMEM((NEG_K * per_w,), jnp.int32),
            pltpu.VMEM((2, CHUNK, 2 * DIM), jnp.float32),
            pltpu.VMEM((2, CHUNK, 2 * DIM), jnp.float32),
            pltpu.VMEM((2, NEG_K * CHUNK, 2 * DIM), jnp.float32),
            pltpu.VMEM((LANES * CHUNK,), jnp.float32),
            pltpu.VMEM((LANES * CHUNK,), jnp.float32),
            pltpu.VMEM((per_w,), jnp.float32),
            pltpu.VMEM((per_w,), jnp.float32),
            pltpu.SemaphoreType.DMA,
            pltpu.SemaphoreType.DMA,
        ],
    )
    def k(u_g_h, vp_g_h, vn_g_h, u_tab, v_tab,
          pos_out, neg_out,
          u_gv, vp_gv, vn_gv, u_b, vp_b, vn_b,
          pscr, nscr, pos_sv, neg_sv, sem0, sem1):
        wid = lax.axis_index("s") * nc + lax.axis_index("c")
        wbase = wid * per_w

        pltpu.sync_copy(u_g_h.at[pl.ds(wbase, per_w)], u_gv)
        pltpu.sync_copy(vp_g_h.at[pl.ds(wbase, per_w)], vp_gv)
        for n in range(NEG_K):
            pltpu.sync_copy(vn_g_h.at[pl.ds(n * B + wbase, per_w)],
                            vn_gv.at[pl.ds(n * per_w, per_w)])

        sems = (sem0, sem1)

        def fire(c):
            p = c % 2
            cb = c * CHUNK
            hs = [
                pltpu.async_copy(
                    u_tab.at[u_gv.at[pl.ds(cb, CHUNK)]], u_b.at[p], sems[p]),
                pltpu.async_copy(
                    v_tab.at[vp_gv.at[pl.ds(cb, CHUNK)]], vp_b.at[p], sems[p]),
            ]
            for n in range(NEG_K):
                hs.append(pltpu.async_copy(
                    v_tab.at[vn_gv.at[pl.ds(n * per_w + cb, CHUNK)]],
                    vn_b.at[p, pl.ds(n * CHUNK, CHUNK)], sems[p]))
            return hs

        pending = fire(0)
        iota16 = lax.iota(jnp.int32, LANES)

        for c in range(n_chunks):
            nxt = fire(c + 1) if c + 1 < n_chunks else []
            for h in pending:
                h.wait()
            pending = nxt
            p = c % 2
            cb = c * CHUNK

            def elem_body(e, carry):
                pos_p = jnp.zeros((LANES,), jnp.float32)
                neg_p = jnp.zeros((LANES,), jnp.float32)
                for j in range(DIM // LANES):
                    sl = pl.ds(j * LANES, LANES)
                    u = u_b[p, e, sl]
                    vsum = vn_b[p, e, sl]
                    for n in range(1, NEG_K):
                        vsum = vsum + vn_b[p, n * CHUNK + e, sl]
                    pos_p = pos_p + u * vp_b[p, e, sl]
                    neg_p = neg_p + u * vsum
                sidx = iota16 * CHUNK + e
                plsc.store_scatter(pscr, [sidx], pos_p)
                plsc.store_scatter(nscr, [sidx], neg_p)
                return carry

            lax.fori_loop(0, CHUNK, elem_body, 0)

            def red_body(g, carry):
                gb = g * LANES
                acc_p = pscr[pl.ds(gb, LANES)]
                acc_n = nscr[pl.ds(gb, LANES)]
                for l in range(1, LANES):
                    acc_p = acc_p + pscr[pl.ds(l * CHUNK + gb, LANES)]
                    acc_n = acc_n + nscr[pl.ds(l * CHUNK + gb, LANES)]
                off = cb + gb
                pos_sv[pl.ds(off, LANES)] = acc_p
                neg_sv[pl.ds(off, LANES)] = acc_n
                return carry

            lax.fori_loop(0, CHUNK // LANES, red_body, 0)

        pltpu.sync_copy(pos_sv, pos_out.at[pl.ds(wbase, per_w)])
        pltpu.sync_copy(neg_sv, neg_out.at[pl.ds(wbase, per_w)])

    return k(u_g, vp_g, vn_g, u_tab2, v_tab2)


def _tc_loss_body(pos_ref, neg_ref, bs_ref, out_ref):
    pos = pos_ref[...]
    neg = neg_ref[...]
    ls = jax.nn.log_sigmoid(pos) + jax.nn.log_sigmoid(-neg)
    out_ref[0, 0] = -jnp.sum(ls) / bs_ref[0].astype(jnp.float32)


def kernel(u_positive, v_positive, v_negative, batch_size, u_table, v_table):
    B = u_positive.shape[0]
    u_idx = u_positive.astype(jnp.int32)
    vp_idx = v_positive.astype(jnp.int32)
    vn_idx = v_negative.astype(jnp.int32).T.reshape((-1,))

    u_tab2 = _pack_table(u_table.T)
    v_tab2 = _pack_table(v_table.T)

    pos_s, neg_s = _sc_scores(u_idx, vp_idx, vn_idx, u_tab2, v_tab2)

    rows = B // 128
    bs = jnp.asarray(batch_size, jnp.int32).reshape((1,))
    loss = pl.pallas_call(
        _tc_loss_body,
        out_shape=jax.ShapeDtypeStruct((1, 1), jnp.float32),
        in_specs=[
            pl.BlockSpec(memory_space=pltpu.VMEM),
            pl.BlockSpec(memory_space=pltpu.VMEM),
            pl.BlockSpec(memory_space=pltpu.SMEM),
        ],
        out_specs=pl.BlockSpec(memory_space=pltpu.SMEM),
    )(pos_s.reshape((rows, 128)), neg_s.reshape((rows, 128)), bs)
    return loss[0, 0]


# speedup vs baseline: 18.2888x; 1.3251x over previous
import functools

import jax
import jax.numpy as jnp
from jax import lax
from jax.experimental import pallas as pl
from jax.experimental.pallas import tpu as pltpu
from jax.experimental.pallas import tpu_sc as plsc

DIM = 64
LANES = 16
CHUNK = 64
NEG_K = 5
PACK_W = 4096


def _tc_pack_body(u_ref, v_ref, out_ref):
    r = lax.broadcasted_iota(jnp.int32, (DIM, DIM), 0)
    c = lax.broadcasted_iota(jnp.int32, (DIM, DIM), 1)
    ident = (r == c).astype(jnp.float32)
    out_ref[:, 0:DIM] = lax.dot_general(
        u_ref[...], ident, (((0,), (0,)), ((), ())),
        preferred_element_type=jnp.float32)
    out_ref[:, DIM:2 * DIM] = lax.dot_general(
        v_ref[...], ident, (((0,), (0,)), ((), ())),
        preferred_element_type=jnp.float32)


def _pack_tables(u_t, v_t):
    V = u_t.shape[1]
    grid = (V + PACK_W - 1) // PACK_W
    return pl.pallas_call(
        _tc_pack_body,
        grid=(grid,),
        in_specs=[
            pl.BlockSpec((DIM, PACK_W), lambda j: (0, j)),
            pl.BlockSpec((DIM, PACK_W), lambda j: (0, j)),
        ],
        out_specs=pl.BlockSpec((PACK_W, 2 * DIM), lambda j: (j, 0)),
        out_shape=jax.ShapeDtypeStruct((V, 2 * DIM), jnp.float32),
    )(u_t, v_t)


def _sc_scores(u_g, vp_g, vn_g, tab2):
    B = u_g.shape[0]
    info = plsc.get_sparse_core_info()
    nc, ns = info.num_cores, info.num_subcores
    nw = nc * ns
    per_w = B // nw
    n_chunks = per_w // CHUNK
    mesh = plsc.VectorSubcoreMesh(core_axis_name="c", subcore_axis_name="s")

    @functools.partial(
        pl.kernel,
        mesh=mesh,
        compiler_params=pltpu.CompilerParams(needs_layout_passes=False),
        out_type=(
            jax.ShapeDtypeStruct((B,), jnp.float32),
            jax.ShapeDtypeStruct((B,), jnp.float32),
        ),
        scratch_types=[
            pltpu.VMEM((per_w,), jnp.int32),
            pltpu.VMEM((per_w,), jnp.int32),
            pltpu.VMEM((NEG_K * per_w,), jnp.int32),
            pltpu.VMEM((2, CHUNK, 2 * DIM), jnp.float32),
            pltpu.VMEM((2, CHUNK, 2 * DIM), jnp.float32),
            pltpu.VMEM((2, NEG_K * CHUNK, 2 * DIM), jnp.float32),
            pltpu.VMEM((LANES * CHUNK,), jnp.float32),
            pltpu.VMEM((LANES * CHUNK,), jnp.float32),
            pltpu.VMEM((per_w,), jnp.float32),
            pltpu.VMEM((per_w,), jnp.float32),
            pltpu.SemaphoreType.DMA,
            pltpu.SemaphoreType.DMA,
        ],
    )
    def k(u_g_h, vp_g_h, vn_g_h, tab,
          pos_out, neg_out,
          u_gv, vp_gv, vn_gv, u_b, vp_b, vn_b,
          pscr, nscr, pos_sv, neg_sv, sem0, sem1):
        wid = lax.axis_index("s") * nc + lax.axis_index("c")
        wbase = wid * per_w

        pltpu.sync_copy(u_g_h.at[pl.ds(wbase, per_w)], u_gv)
        pltpu.sync_copy(vp_g_h.at[pl.ds(wbase, per_w)], vp_gv)
        for n in range(NEG_K):
            pltpu.sync_copy(vn_g_h.at[pl.ds(n * B + wbase, per_w)],
                            vn_gv.at[pl.ds(n * per_w, per_w)])

        sems = (sem0, sem1)

        def fire(c):
            p = c % 2
            cb = c * CHUNK
            hs = [
                pltpu.async_copy(
                    tab.at[u_gv.at[pl.ds(cb, CHUNK)]], u_b.at[p], sems[p]),
                pltpu.async_copy(
                    tab.at[vp_gv.at[pl.ds(cb, CHUNK)]], vp_b.at[p], sems[p]),
            ]
            for n in range(NEG_K):
                hs.append(pltpu.async_copy(
                    tab.at[vn_gv.at[pl.ds(n * per_w + cb, CHUNK)]],
                    vn_b.at[p, pl.ds(n * CHUNK, CHUNK)], sems[p]))
            return hs

        pending = fire(0)
        iota16 = lax.iota(jnp.int32, LANES)

        for c in range(n_chunks):
            nxt = fire(c + 1) if c + 1 < n_chunks else []
            for h in pending:
                h.wait()
            pending = nxt
            p = c % 2
            cb = c * CHUNK

            def elem_body(e, carry):
                pos_p = jnp.zeros((LANES,), jnp.float32)
                neg_p = jnp.zeros((LANES,), jnp.float32)
                for j in range(DIM // LANES):
                    sl = pl.ds(j * LANES, LANES)
                    sv = pl.ds(DIM + j * LANES, LANES)
                    u = u_b[p, e, sl]
                    vsum = vn_b[p, e, sv]
                    for n in range(1, NEG_K):
                        vsum = vsum + vn_b[p, n * CHUNK + e, sv]
                    pos_p = pos_p + u * vp_b[p, e, sv]
                    neg_p = neg_p + u * vsum
                sidx = iota16 * CHUNK + e
                plsc.store_scatter(pscr, [sidx], pos_p)
                plsc.store_scatter(nscr, [sidx], neg_p)
                return carry

            lax.fori_loop(0, CHUNK, elem_body, 0)

            def red_body(g, carry):
                gb = g * LANES
                acc_p = pscr[pl.ds(gb, LANES)]
                acc_n = nscr[pl.ds(gb, LANES)]
                for l in range(1, LANES):
                    acc_p = acc_p + pscr[pl.ds(l * CHUNK + gb, LANES)]
                    acc_n = acc_n + nscr[pl.ds(l * CHUNK + gb, LANES)]
                off = cb + gb
                pos_sv[pl.ds(off, LANES)] = acc_p
                neg_sv[pl.ds(off, LANES)] = acc_n
                return carry

            lax.fori_loop(0, CHUNK // LANES, red_body, 0)

        pltpu.sync_copy(pos_sv, pos_out.at[pl.ds(wbase, per_w)])
        pltpu.sync_copy(neg_sv, neg_out.at[pl.ds(wbase, per_w)])

    return k(u_g, vp_g, vn_g, tab2)


def _tc_loss_body(pos_ref, neg_ref, bs_ref, out_ref):
    pos = pos_ref[...]
    neg = neg_ref[...]
    ls = jax.nn.log_sigmoid(pos) + jax.nn.log_sigmoid(-neg)
    out_ref[0, 0] = -jnp.sum(ls) / bs_ref[0].astype(jnp.float32)


def kernel(u_positive, v_positive, v_negative, batch_size, u_table, v_table):
    B = u_positive.shape[0]
    u_idx = u_positive.astype(jnp.int32)
    vp_idx = v_positive.astype(jnp.int32)
    vn_idx = v_negative.astype(jnp.int32).T.reshape((-1,))

    tab2 = _pack_tables(u_table.T, v_table.T)

    pos_s, neg_s = _sc_scores(u_idx, vp_idx, vn_idx, tab2)

    rows = B // 128
    bs = jnp.asarray(batch_size, jnp.int32).reshape((1,))
    loss = pl.pallas_call(
        _tc_loss_body,
        out_shape=jax.ShapeDtypeStruct((1, 1), jnp.float32),
        in_specs=[
            pl.BlockSpec(memory_space=pltpu.VMEM),
            pl.BlockSpec(memory_space=pltpu.VMEM),
            pl.BlockSpec(memory_space=pltpu.SMEM),
        ],
        out_specs=pl.BlockSpec(memory_space=pltpu.SMEM),
    )(pos_s.reshape((rows, 128)), neg_s.reshape((rows, 128)), bs)
    return loss[0, 0]


# speedup vs baseline: 20.7917x; 1.1369x over previous
import functools

import jax
import jax.numpy as jnp
from jax import lax
from jax.experimental import pallas as pl
from jax.experimental.pallas import tpu as pltpu
from jax.experimental.pallas import tpu_sc as plsc

DIM = 64
LANES = 16
CHUNK = 64
NEG_K = 5
PACK_W = 8192


def _tc_pack_body(u_ref, v_ref, out_ref):
    r = lax.broadcasted_iota(jnp.int32, (DIM, DIM), 0)
    c = lax.broadcasted_iota(jnp.int32, (DIM, DIM), 1)
    ident = (r == c).astype(jnp.float32)
    out_ref[:, 0:DIM] = lax.dot_general(
        u_ref[...], ident, (((0,), (0,)), ((), ())),
        preferred_element_type=jnp.float32)
    out_ref[:, DIM:2 * DIM] = lax.dot_general(
        v_ref[...], ident, (((0,), (0,)), ((), ())),
        preferred_element_type=jnp.float32)


def _pack_tables(u_t, v_t):
    V = u_t.shape[1]
    grid = (V + PACK_W - 1) // PACK_W
    return pl.pallas_call(
        _tc_pack_body,
        grid=(grid,),
        in_specs=[
            pl.BlockSpec((DIM, PACK_W), lambda j: (0, j)),
            pl.BlockSpec((DIM, PACK_W), lambda j: (0, j)),
        ],
        out_specs=pl.BlockSpec((PACK_W, 2 * DIM), lambda j: (j, 0)),
        out_shape=jax.ShapeDtypeStruct((V, 2 * DIM), jnp.float32),
    )(u_t, v_t)


def _sc_scores(u_g, vp_g, vn_g, tab2):
    B = u_g.shape[0]
    info = plsc.get_sparse_core_info()
    nc, ns = info.num_cores, info.num_subcores
    nw = nc * ns
    per_w = B // nw
    n_chunks = per_w // CHUNK
    mesh = plsc.VectorSubcoreMesh(core_axis_name="c", subcore_axis_name="s")

    @functools.partial(
        pl.kernel,
        mesh=mesh,
        compiler_params=pltpu.CompilerParams(needs_layout_passes=False),
        out_type=(
            jax.ShapeDtypeStruct((B,), jnp.float32),
            jax.ShapeDtypeStruct((B,), jnp.float32),
        ),
        scratch_types=[
            pltpu.VMEM((per_w,), jnp.int32),
            pltpu.VMEM((per_w,), jnp.int32),
            pltpu.VMEM((NEG_K * per_w,), jnp.int32),
            pltpu.VMEM((2, CHUNK, 2 * DIM), jnp.float32),
            pltpu.VMEM((2, CHUNK, 2 * DIM), jnp.float32),
            pltpu.VMEM((2, NEG_K * CHUNK, 2 * DIM), jnp.float32),
            pltpu.VMEM((LANES * CHUNK,), jnp.float32),
            pltpu.VMEM((LANES * CHUNK,), jnp.float32),
            pltpu.VMEM((per_w,), jnp.float32),
            pltpu.VMEM((per_w,), jnp.float32),
            pltpu.SemaphoreType.DMA,
            pltpu.SemaphoreType.DMA,
        ],
    )
    def k(u_g_h, vp_g_h, vn_g_h, tab,
          pos_out, neg_out,
          u_gv, vp_gv, vn_gv, u_b, vp_b, vn_b,
          pscr, nscr, pos_sv, neg_sv, sem0, sem1):
        wid = lax.axis_index("s") * nc + lax.axis_index("c")
        wbase = wid * per_w

        pltpu.sync_copy(u_g_h.at[pl.ds(wbase, per_w)], u_gv)
        pltpu.sync_copy(vp_g_h.at[pl.ds(wbase, per_w)], vp_gv)
        for n in range(NEG_K):
            pltpu.sync_copy(vn_g_h.at[pl.ds(n * B + wbase, per_w)],
                            vn_gv.at[pl.ds(n * per_w, per_w)])

        sems = (sem0, sem1)

        def fire(c):
            p = c % 2
            cb = c * CHUNK
            hs = [
                pltpu.async_copy(
                    tab.at[u_gv.at[pl.ds(cb, CHUNK)]], u_b.at[p], sems[p]),
                pltpu.async_copy(
                    tab.at[vp_gv.at[pl.ds(cb, CHUNK)]], vp_b.at[p], sems[p]),
            ]
            for n in range(NEG_K):
                hs.append(pltpu.async_copy(
                    tab.at[vn_gv.at[pl.ds(n * per_w + cb, CHUNK)]],
                    vn_b.at[p, pl.ds(n * CHUNK, CHUNK)], sems[p]))
            return hs

        pending = fire(0)
        iota16 = lax.iota(jnp.int32, LANES)

        for c in range(n_chunks):
            nxt = fire(c + 1) if c + 1 < n_chunks else []
            for h in pending:
                h.wait()
            pending = nxt
            p = c % 2
            cb = c * CHUNK

            def elem_body(e, carry):
                pos_p = jnp.zeros((LANES,), jnp.float32)
                neg_p = jnp.zeros((LANES,), jnp.float32)
                for j in range(DIM // LANES):
                    sl = pl.ds(j * LANES, LANES)
                    sv = pl.ds(DIM + j * LANES, LANES)
                    u = u_b[p, e, sl]
                    vsum = vn_b[p, e, sv]
                    for n in range(1, NEG_K):
                        vsum = vsum + vn_b[p, n * CHUNK + e, sv]
                    pos_p = pos_p + u * vp_b[p, e, sv]
                    neg_p = neg_p + u * vsum
                sidx = iota16 * CHUNK + e
                plsc.store_scatter(pscr, [sidx], pos_p)
                plsc.store_scatter(nscr, [sidx], neg_p)
                return carry

            lax.fori_loop(0, CHUNK, elem_body, 0)

            def red_body(g, carry):
                gb = g * LANES
                acc_p = pscr[pl.ds(gb, LANES)]
                acc_n = nscr[pl.ds(gb, LANES)]
                for l in range(1, LANES):
                    acc_p = acc_p + pscr[pl.ds(l * CHUNK + gb, LANES)]
                    acc_n = acc_n + nscr[pl.ds(l * CHUNK + gb, LANES)]
                off = cb + gb
                pos_sv[pl.ds(off, LANES)] = acc_p
                neg_sv[pl.ds(off, LANES)] = acc_n
                return carry

            lax.fori_loop(0, CHUNK // LANES, red_body, 0)

        pltpu.sync_copy(pos_sv, pos_out.at[pl.ds(wbase, per_w)])
        pltpu.sync_copy(neg_sv, neg_out.at[pl.ds(wbase, per_w)])

    return k(u_g, vp_g, vn_g, tab2)


def _tc_loss_body(pos_ref, neg_ref, bs_ref, out_ref):
    pos = pos_ref[...]
    neg = neg_ref[...]
    ls = jax.nn.log_sigmoid(pos) + jax.nn.log_sigmoid(-neg)
    out_ref[0, 0] = -jnp.sum(ls) / bs_ref[0].astype(jnp.float32)


def kernel(u_positive, v_positive, v_negative, batch_size, u_table, v_table):
    B = u_positive.shape[0]
    u_idx = u_positive.astype(jnp.int32)
    vp_idx = v_positive.astype(jnp.int32)
    vn_idx = v_negative.astype(jnp.int32).T.reshape((-1,))

    tab2 = _pack_tables(u_table.T, v_table.T)

    pos_s, neg_s = _sc_scores(u_idx, vp_idx, vn_idx, tab2)

    rows = B // 128
    bs = jnp.asarray(batch_size, jnp.int32).reshape((1,))
    loss = pl.pallas_call(
        _tc_loss_body,
        out_shape=jax.ShapeDtypeStruct((1, 1), jnp.float32),
        in_specs=[
            pl.BlockSpec(memory_space=pltpu.VMEM),
            pl.BlockSpec(memory_space=pltpu.VMEM),
            pl.BlockSpec(memory_space=pltpu.SMEM),
        ],
        out_specs=pl.BlockSpec(memory_space=pltpu.SMEM),
    )(pos_s.reshape((rows, 128)), neg_s.reshape((rows, 128)), bs)
    return loss[0, 0]


# speedup vs baseline: 22.1065x; 1.0632x over previous
import functools

import jax
import jax.numpy as jnp
from jax import lax
from jax.experimental import pallas as pl
from jax.experimental.pallas import tpu as pltpu
from jax.experimental.pallas import tpu_sc as plsc

DIM = 64
LANES = 16
CHUNK = 64
NEG_K = 5
PACK_W = 16384


def _tc_pack_body(u_ref, v_ref, out_ref):
    r = lax.broadcasted_iota(jnp.int32, (DIM, DIM), 0)
    c = lax.broadcasted_iota(jnp.int32, (DIM, DIM), 1)
    ident = (r == c).astype(jnp.float32)
    out_ref[:, 0:DIM] = lax.dot_general(
        u_ref[...], ident, (((0,), (0,)), ((), ())),
        preferred_element_type=jnp.float32)
    out_ref[:, DIM:2 * DIM] = lax.dot_general(
        v_ref[...], ident, (((0,), (0,)), ((), ())),
        preferred_element_type=jnp.float32)


def _pack_tables(u_t, v_t):
    V = u_t.shape[1]
    grid = (V + PACK_W - 1) // PACK_W
    return pl.pallas_call(
        _tc_pack_body,
        grid=(grid,),
        in_specs=[
            pl.BlockSpec((DIM, PACK_W), lambda j: (0, j)),
            pl.BlockSpec((DIM, PACK_W), lambda j: (0, j)),
        ],
        out_specs=pl.BlockSpec((PACK_W, 2 * DIM), lambda j: (j, 0)),
        out_shape=jax.ShapeDtypeStruct((V, 2 * DIM), jnp.float32),
    )(u_t, v_t)


def _sc_scores(u_g, vp_g, vn_g, tab2):
    B = u_g.shape[0]
    info = plsc.get_sparse_core_info()
    nc, ns = info.num_cores, info.num_subcores
    nw = nc * ns
    per_w = B // nw
    n_chunks = per_w // CHUNK
    mesh = plsc.VectorSubcoreMesh(core_axis_name="c", subcore_axis_name="s")

    @functools.partial(
        pl.kernel,
        mesh=mesh,
        compiler_params=pltpu.CompilerParams(needs_layout_passes=False),
        out_type=(
            jax.ShapeDtypeStruct((B,), jnp.float32),
            jax.ShapeDtypeStruct((B,), jnp.float32),
        ),
        scratch_types=[
            pltpu.VMEM((per_w,), jnp.int32),
            pltpu.VMEM((per_w,), jnp.int32),
            pltpu.VMEM((NEG_K * per_w,), jnp.int32),
            pltpu.VMEM((2, CHUNK, 2 * DIM), jnp.float32),
            pltpu.VMEM((2, CHUNK, 2 * DIM), jnp.float32),
            pltpu.VMEM((2, NEG_K * CHUNK, 2 * DIM), jnp.float32),
            pltpu.VMEM((LANES * CHUNK,), jnp.float32),
            pltpu.VMEM((LANES * CHUNK,), jnp.float32),
            pltpu.VMEM((per_w,), jnp.float32),
            pltpu.VMEM((per_w,), jnp.float32),
            pltpu.SemaphoreType.DMA,
            pltpu.SemaphoreType.DMA,
        ],
    )
    def k(u_g_h, vp_g_h, vn_g_h, tab,
          pos_out, neg_out,
          u_gv, vp_gv, vn_gv, u_b, vp_b, vn_b,
          pscr, nscr, pos_sv, neg_sv, sem0, sem1):
        wid = lax.axis_index("s") * nc + lax.axis_index("c")
        wbase = wid * per_w

        pltpu.sync_copy(u_g_h.at[pl.ds(wbase, per_w)], u_gv)
        pltpu.sync_copy(vp_g_h.at[pl.ds(wbase, per_w)], vp_gv)
        for n in range(NEG_K):
            pltpu.sync_copy(vn_g_h.at[pl.ds(n * B + wbase, per_w)],
                            vn_gv.at[pl.ds(n * per_w, per_w)])

        sems = (sem0, sem1)

        def fire(c):
            p = c % 2
            cb = c * CHUNK
            hs = [
                pltpu.async_copy(
                    tab.at[u_gv.at[pl.ds(cb, CHUNK)]], u_b.at[p], sems[p]),
                pltpu.async_copy(
                    tab.at[vp_gv.at[pl.ds(cb, CHUNK)]], vp_b.at[p], sems[p]),
            ]
            for n in range(NEG_K):
                hs.append(pltpu.async_copy(
                    tab.at[vn_gv.at[pl.ds(n * per_w + cb, CHUNK)]],
                    vn_b.at[p, pl.ds(n * CHUNK, CHUNK)], sems[p]))
            return hs

        pending = fire(0)
        iota16 = lax.iota(jnp.int32, LANES)

        for c in range(n_chunks):
            nxt = fire(c + 1) if c + 1 < n_chunks else []
            for h in pending:
                h.wait()
            pending = nxt
            p = c % 2
            cb = c * CHUNK

            def elem_body(e, carry):
                pos_p = jnp.zeros((LANES,), jnp.float32)
                neg_p = jnp.zeros((LANES,), jnp.float32)
                for j in range(DIM // LANES):
                    sl = pl.ds(j * LANES, LANES)
                    sv = pl.ds(DIM + j * LANES, LANES)
                    u = u_b[p, e, sl]
                    vsum = vn_b[p, e, sv]
                    for n in range(1, NEG_K):
                        vsum = vsum + vn_b[p, n * CHUNK + e, sv]
                    pos_p = pos_p + u * vp_b[p, e, sv]
                    neg_p = neg_p + u * vsum
                sidx = iota16 * CHUNK + e
                plsc.store_scatter(pscr, [sidx], pos_p)
                plsc.store_scatter(nscr, [sidx], neg_p)
                return carry

            lax.fori_loop(0, CHUNK, elem_body, 0)

            def red_body(g, carry):
                gb = g * LANES
                acc_p = pscr[pl.ds(gb, LANES)]
                acc_n = nscr[pl.ds(gb, LANES)]
                for l in range(1, LANES):
                    acc_p = acc_p + pscr[pl.ds(l * CHUNK + gb, LANES)]
                    acc_n = acc_n + nscr[pl.ds(l * CHUNK + gb, LANES)]
                off = cb + gb
                pos_sv[pl.ds(off, LANES)] = acc_p
                neg_sv[pl.ds(off, LANES)] = acc_n
                return carry

            lax.fori_loop(0, CHUNK // LANES, red_body, 0)

        pltpu.sync_copy(pos_sv, pos_out.at[pl.ds(wbase, per_w)])
        pltpu.sync_copy(neg_sv, neg_out.at[pl.ds(wbase, per_w)])

    return k(u_g, vp_g, vn_g, tab2)


def _tc_loss_body(pos_ref, neg_ref, bs_ref, out_ref):
    pos = pos_ref[...]
    neg = neg_ref[...]
    ls = jax.nn.log_sigmoid(pos) + jax.nn.log_sigmoid(-neg)
    out_ref[0, 0] = -jnp.sum(ls) / bs_ref[0].astype(jnp.float32)


def kernel(u_positive, v_positive, v_negative, batch_size, u_table, v_table):
    B = u_positive.shape[0]
    u_idx = u_positive.astype(jnp.int32)
    vp_idx = v_positive.astype(jnp.int32)
    vn_idx = v_negative.astype(jnp.int32).T.reshape((-1,))

    tab2 = _pack_tables(u_table.T, v_table.T)

    pos_s, neg_s = _sc_scores(u_idx, vp_idx, vn_idx, tab2)

    rows = B // 128
    bs = jnp.asarray(batch_size, jnp.int32).reshape((1,))
    loss = pl.pallas_call(
        _tc_loss_body,
        out_shape=jax.ShapeDtypeStruct((1, 1), jnp.float32),
        in_specs=[
            pl.BlockSpec(memory_space=pltpu.VMEM),
            pl.BlockSpec(memory_space=pltpu.VMEM),
            pl.BlockSpec(memory_space=pltpu.SMEM),
        ],
        out_specs=pl.BlockSpec(memory_space=pltpu.SMEM),
    )(pos_s.reshape((rows, 128)), neg_s.reshape((rows, 128)), bs)
    return loss[0, 0]
